# baseline (speedup 1.0000x reference)
import functools

import jax
import jax.numpy as jnp
from jax import lax
from jax.experimental import pallas as pl
from jax.experimental.pallas import tpu as pltpu
from jax.experimental.pallas import tpu_sc as plsc

NC = 2
NS = 16
NW = NC * NS
EB = 96
KCH = 16
BN = 512
F16 = 16
NBUF = 2


def _cdiv(a, b):
    return (a + b - 1) // b


def _sc_worker_id():
    return lax.axis_index("s") * NC + lax.axis_index("c")


def _sc_zero_table(zsh, zeros_hbm, s):
    @pl.when(s == 0)
    def _():
        pltpu.sync_copy(zeros_hbm, zsh)


def _sc_copy_out(zsh, out_hbm, c, s):
    @pl.when(s == 0)
    def _():
        pltpu.sync_copy(zsh, out_hbm.at[c])


@functools.cache
def _make_deg_kernel(NPAD, EPR):
    RW = EPR // NW
    NCHUNK = RW // KCH
    mesh = plsc.VectorSubcoreMesh(core_axis_name="c", subcore_axis_name="s")

    @functools.partial(
        pl.kernel,
        out_type=jax.ShapeDtypeStruct((NC, NPAD, F16), jnp.float32),
        mesh=mesh,
        scratch_types=[
            pltpu.VMEM((KCH, EB), jnp.int32),
            pltpu.VMEM((EB, F16), jnp.float32),
            pltpu.VMEM_SHARED((NPAD, F16), jnp.float32),
            pltpu.SemaphoreType.DMA,
        ],
        compiler_params=pltpu.CompilerParams(use_tc_tiling_on_sc=False),
    )
    def deg_kernel(dst_hbm, ones_hbm, zeros_hbm, out_hbm, dstv, onesv, zsh,
                   ssem):
        c = lax.axis_index("c")
        s = lax.axis_index("s")
        w = _sc_worker_id()
        pltpu.sync_copy(ones_hbm, onesv)
        _sc_zero_table(zsh, zeros_hbm, s)
        plsc.subcore_barrier()

        base = w * RW

        def chunk(m, carry):
            r0 = base + m * KCH
            pltpu.sync_copy(dst_hbm.at[pl.ds(r0, KCH)], dstv)
            hs = [pltpu.async_copy(onesv, zsh.at[dstv.at[j]], ssem, add=True)
                  for j in range(KCH)]
            for h in hs:
                h.wait()
            return carry
        lax.fori_loop(0, NCHUNK, chunk, 0)

        plsc.subcore_barrier()
        _sc_copy_out(zsh, out_hbm, c, s)

    return deg_kernel


@functools.cache
def _make_prop_kernel(NPAD, EPR):
    RW = EPR // NW
    NCHUNK = RW // KCH
    mesh = plsc.VectorSubcoreMesh(core_axis_name="c", subcore_axis_name="s")

    @functools.partial(
        pl.kernel,
        out_type=jax.ShapeDtypeStruct((NC, NPAD, F16), jnp.float32),
        mesh=mesh,
        scratch_types=[
            pltpu.VMEM((KCH, EB), jnp.int32),
            pltpu.VMEM((KCH, EB), jnp.int32),
            pltpu.VMEM((NBUF, EB, F16), jnp.float32),
            pltpu.VMEM_SHARED((NPAD, F16), jnp.float32),
            pltpu.SemaphoreType.DMA,
            pltpu.SemaphoreType.DMA,
        ],
        compiler_params=pltpu.CompilerParams(use_tc_tiling_on_sc=False),
    )
    def prop_kernel(y_hbm, src_hbm, dst_hbm, zeros_hbm, out_hbm,
                    srcv, dstv, rows, zsh, gsem, ssem):
        c = lax.axis_index("c")
        s = lax.axis_index("s")
        w = _sc_worker_id()
        _sc_zero_table(zsh, zeros_hbm, s)
        plsc.subcore_barrier()

        base = w * RW

        def chunk(m, carry):
            r0 = base + m * KCH
            pltpu.sync_copy(src_hbm.at[pl.ds(r0, KCH)], srcv)
            pltpu.sync_copy(dst_hbm.at[pl.ds(r0, KCH)], dstv)
            gp = [None] * NBUF
            sp = [None] * NBUF
            for j in range(KCH):
                b = j % NBUF
                if sp[b] is not None:
                    sp[b].wait()
                    sp[b] = None
                gp[b] = pltpu.async_copy(
                    y_hbm.at[srcv.at[j]], rows.at[b], gsem)
                pb = (j - 1) % NBUF
                if j > 0:
                    gp[pb].wait()
                    gp[pb] = None
                    sp[pb] = pltpu.async_copy(
                        rows.at[pb], zsh.at[dstv.at[j - 1]], ssem, add=True)
            lb = (KCH - 1) % NBUF
            gp[lb].wait()
            sp[lb] = pltpu.async_copy(
                rows.at[lb], zsh.at[dstv.at[KCH - 1]], ssem, add=True)
            for b in range(NBUF):
                if sp[b] is not None:
                    sp[b].wait()
            return carry
        lax.fori_loop(0, NCHUNK, chunk, 0)

        plsc.subcore_barrier()
        _sc_copy_out(zsh, out_hbm, c, s)

    return prop_kernel


def _edge_pipeline(y_hbm, src_hbm, dst_hbm, zsh, srcv, dstv, rows,
                   gsem, ssem, base, NCHUNK):
    def chunk(m, carry):
        r0 = base + m * KCH
        pltpu.sync_copy(src_hbm.at[pl.ds(r0, KCH)], srcv)
        pltpu.sync_copy(dst_hbm.at[pl.ds(r0, KCH)], dstv)
        gp = [None] * NBUF
        sp = [None] * NBUF
        for j in range(KCH):
            b = j % NBUF
            if sp[b] is not None:
                sp[b].wait()
                sp[b] = None
            gp[b] = pltpu.async_copy(y_hbm.at[srcv.at[j]], rows.at[b], gsem)
            pb = (j - 1) % NBUF
            if j > 0:
                gp[pb].wait()
                gp[pb] = None
                sp[pb] = pltpu.async_copy(
                    rows.at[pb], zsh.at[dstv.at[j - 1]], ssem, add=True)
        lb = (KCH - 1) % NBUF
        gp[lb].wait()
        sp[lb] = pltpu.async_copy(
            rows.at[lb], zsh.at[dstv.at[KCH - 1]], ssem, add=True)
        for b in range(NBUF):
            if sp[b] is not None:
                sp[b].wait()
        return carry
    lax.fori_loop(0, NCHUNK, chunk, 0)


@functools.cache
def _make_dualprop_kernel(NPAD, EPR):
    RW = EPR // NS
    NCHUNK = RW // KCH
    mesh = plsc.VectorSubcoreMesh(core_axis_name="c", subcore_axis_name="s")

    @functools.partial(
        pl.kernel,
        out_type=[jax.ShapeDtypeStruct((NPAD, F16), jnp.float32),
                  jax.ShapeDtypeStruct((NPAD, F16), jnp.float32)],
        mesh=mesh,
        scratch_types=[
            pltpu.VMEM((KCH, EB), jnp.int32),
            pltpu.VMEM((KCH, EB), jnp.int32),
            pltpu.VMEM((NBUF, EB, F16), jnp.float32),
            pltpu.VMEM_SHARED((NPAD, F16), jnp.float32),
            pltpu.SemaphoreType.DMA,
            pltpu.SemaphoreType.DMA,
        ],
        compiler_params=pltpu.CompilerParams(use_tc_tiling_on_sc=False),
    )
    def dual_kernel(ya_hbm, yb_hbm, src_hbm, dst_hbm, zeros_hbm,
                    outa_hbm, outb_hbm, srcv, dstv, rows, zsh, gsem, ssem):
        c = lax.axis_index("c")
        s = lax.axis_index("s")
        _sc_zero_table(zsh, zeros_hbm, s)
        plsc.subcore_barrier()

        base = s * RW

        @pl.when(c == 0)
        def _():
            _edge_pipeline(ya_hbm, src_hbm, dst_hbm, zsh, srcv, dstv, rows,
                           gsem, ssem, base, NCHUNK)

        @pl.when(c == 1)
        def _():
            _edge_pipeline(yb_hbm, src_hbm, dst_hbm, zsh, srcv, dstv, rows,
                           gsem, ssem, base, NCHUNK)

        plsc.subcore_barrier()

        @pl.when(jnp.logical_and(s == 0, c == 0))
        def _():
            pltpu.sync_copy(zsh, outa_hbm)

        @pl.when(jnp.logical_and(s == 0, c == 1))
        def _():
            pltpu.sync_copy(zsh, outb_hbm)

    return dual_kernel


def _dinv_from_deg(degp):
    deg = degp[0, :, 0:1] + degp[1, :, 0:1] + 1.0
    return lax.rsqrt(deg)


def _prep_body(degp_ref, x_ref, ya_ref, yb_ref):
    dinv = _dinv_from_deg(degp_ref[...])
    y = x_ref[...] * dinv
    ya_ref[...] = y[:, :F16]
    yb_ref[...] = y[:, F16:]


def _stage_b_body(z1a_ref, z1b_ref, y1a_ref, y1b_ref, degp_ref,
                  w1_ref, b1_ref, w2_ref, *out_refs):
    dinv = _dinv_from_deg(degp_ref[...])
    u1a = dinv * (z1a_ref[...] + y1a_ref[...])
    u1b = dinv * (z1b_ref[...] + y1b_ref[...])
    u1 = jnp.concatenate([u1a, u1b], axis=1)
    h1 = jax.nn.relu(jnp.dot(u1, w1_ref[...],
                             preferred_element_type=jnp.float32) + b1_ref[...])
    g = jnp.dot(h1, w2_ref[...], preferred_element_type=jnp.float32)
    y2 = g * dinv
    for j, oref in enumerate(out_refs):
        oref[...] = y2[:, F16 * j:F16 * (j + 1)]


def _stage_c_body(*refs):
    (z0, z1, z2, z3, z4, y0, y1, y2, y3, y4, degp_ref,
     b2_ref, wih_ref, whh_ref, bg_ref, wfc_ref, bfc_ref, out_ref) = refs
    dinv = _dinv_from_deg(degp_ref[...])
    parts = []
    for zr, yr in ((z0, y0), (z1, y1), (z2, y2), (z3, y3), (z4, y4)):
        parts.append(dinv * (zr[...] + yr[...]))
    u2 = jnp.concatenate(parts, axis=1)
    gnn = jax.nn.relu(u2 + b2_ref[...])
    whh = whh_ref[...]
    bg = bg_ref[...]
    n = gnn.shape[0]
    gates_x = jnp.dot(gnn, wih_ref[...],
                      preferred_element_type=jnp.float32)
    h = jnp.zeros((n, 16), jnp.float32)
    c = jnp.zeros((n, 16), jnp.float32)
    for t in range(10):
        gates = (gates_x[:, 64 * t:64 * t + 64]
                 + jnp.dot(h, whh, preferred_element_type=jnp.float32) + bg)
        i = jax.nn.sigmoid(gates[:, 0:16])
        f = jax.nn.sigmoid(gates[:, 16:32])
        g = jnp.tanh(gates[:, 32:48])
        o = jax.nn.sigmoid(gates[:, 48:64])
        c = f * c + i * g
        h = o * jnp.tanh(c)
    pred = jnp.sum(h * wfc_ref[...], axis=1, keepdims=True) + bfc_ref[...][:, 0:1]
    out_ref[...] = pred


def _node_spec(width):
    return pl.BlockSpec((BN, width), lambda i: (i, 0))


def _part_spec():
    return pl.BlockSpec((NC, BN, F16), lambda i: (0, i, 0))


def _full_spec(shape):
    return pl.BlockSpec(shape, lambda i: tuple(0 for _ in shape))


@functools.cache
def _make_prep_call(NPAD):
    return pl.pallas_call(
        _prep_body,
        grid=(NPAD // BN,),
        in_specs=[_part_spec(), _node_spec(32)],
        out_specs=[_node_spec(F16), _node_spec(F16)],
        out_shape=[jax.ShapeDtypeStruct((NPAD, F16), jnp.float32)] * 2,
    )


@functools.cache
def _make_stage_b_call(NPAD):
    return pl.pallas_call(
        _stage_b_body,
        grid=(NPAD // BN,),
        in_specs=[_node_spec(F16), _node_spec(F16),
                  _node_spec(F16), _node_spec(F16), _part_spec(),
                  _full_spec((32, 160)), _full_spec((1, 160)),
                  _full_spec((160, 80))],
        out_specs=[_node_spec(F16)] * 5,
        out_shape=[jax.ShapeDtypeStruct((NPAD, F16), jnp.float32)] * 5,
    )


@functools.cache
def _make_stage_c_call(NPAD):
    return pl.pallas_call(
        _stage_c_body,
        grid=(NPAD // BN,),
        in_specs=[_node_spec(F16)] * 5 + [_node_spec(F16)] * 5
        + [_part_spec(),
           _full_spec((1, 80)), _full_spec((80, 640)), _full_spec((16, 64)),
           _full_spec((1, 64)), _full_spec((1, 16)), _full_spec((1, 16))],
        out_specs=_node_spec(1),
        out_shape=jax.ShapeDtypeStruct((NPAD, 1), jnp.float32),
    )


def kernel(x_seq, edge_index, W1, b1, W2, b2, W_ih, W_hh, b_ih, b_hh,
           W_fc, b_fc):
    T, N, FIN = x_seq.shape
    E = edge_index.shape[1]
    NPAD = _cdiv(N + 1, NS * 128) * NS * 128
    EPG = EB * KCH * NW
    EPR = _cdiv(E, EPG) * KCH * NW
    EP = EPR * EB

    src = jnp.concatenate(
        [edge_index[0], jnp.zeros((EP - E,), jnp.int32)]).reshape(EPR, EB)
    dst = jnp.concatenate(
        [edge_index[1], jnp.full((EP - E,), N, jnp.int32)]).reshape(EPR, EB)
    ones_t = jnp.ones((EB, F16), jnp.float32)
    zeros_t = jnp.zeros((NPAD, F16), jnp.float32)

    degp = _make_deg_kernel(NPAD, EPR)(dst, ones_t, zeros_t)

    x2 = jnp.transpose(x_seq, (1, 0, 2)).reshape(N, T * FIN)
    x2p = jnp.pad(x2, ((0, NPAD - N), (0, 32 - T * FIN)))

    y1a, y1b = _make_prep_call(NPAD)(degp, x2p)

    propk = _make_prop_kernel(NPAD, EPR)
    dualk = _make_dualprop_kernel(NPAD, EPR)

    def chain(z_prev):
        return zeros_t + z_prev.reshape(-1)[0] * 0.0

    z1a, z1b = dualk(y1a, y1b, src, dst, chain(degp))

    W1big = jax.scipy.linalg.block_diag(*([W1] * T))
    W1big = jnp.pad(W1big, ((0, 32 - T * FIN), (0, 0)))
    b1big = jnp.tile(b1, T)[None, :]
    W2big = jax.scipy.linalg.block_diag(*([W2] * T))

    y2s = _make_stage_b_call(NPAD)(z1a, z1b, y1a, y1b, degp,
                                   W1big, b1big, W2big)

    z2_0, z2_1 = dualk(y2s[0], y2s[1], src, dst, chain(z1a))
    z2_2, z2_3 = dualk(y2s[2], y2s[3], src, dst, chain(z2_0))
    z2_4p = propk(y2s[4], src, dst, chain(z2_2))
    z2_4 = z2_4p[0] + z2_4p[1]
    z2s = [z2_0, z2_1, z2_2, z2_3, z2_4]

    b2big = jnp.tile(b2, T)[None, :]
    bg = (b_ih + b_hh)[None, :]
    Wihbig = jax.scipy.linalg.block_diag(*([W_ih.T] * T))
    wfc_row = jnp.broadcast_to(W_fc.reshape(1, 16), (1, 16))
    bfc = jnp.broadcast_to(b_fc.reshape(1, 1), (1, 16))

    pred_full = _make_stage_c_call(NPAD)(
        *z2s, *y2s, degp, b2big, Wihbig, W_hh.T, bg, wfc_row, bfc)
    return pred_full[:N]


# speedup vs baseline: 1.0161x; 1.0161x over previous
import functools

import jax
import jax.numpy as jnp
from jax import lax
from jax.experimental import pallas as pl
from jax.experimental.pallas import tpu as pltpu
from jax.experimental.pallas import tpu_sc as plsc

NC = 2
NS = 16
NW = NC * NS
EB = 96
KCH = 16
BN = 512
F16 = 16
NBUF = 2


def _cdiv(a, b):
    return (a + b - 1) // b


def _sc_worker_id():
    return lax.axis_index("s") * NC + lax.axis_index("c")


def _sc_zero_table(zsh, zeros_hbm, s):
    @pl.when(s == 0)
    def _():
        pltpu.sync_copy(zeros_hbm, zsh)


def _sc_copy_out(zsh, out_hbm, c, s):
    @pl.when(s == 0)
    def _():
        pltpu.sync_copy(zsh, out_hbm.at[c])


@functools.cache
def _make_deg_kernel(NPAD, EPR):
    RW = EPR // NW
    NCHUNK = RW // KCH
    mesh = plsc.VectorSubcoreMesh(core_axis_name="c", subcore_axis_name="s")

    @functools.partial(
        pl.kernel,
        out_type=jax.ShapeDtypeStruct((NC, NPAD, F16), jnp.float32),
        mesh=mesh,
        scratch_types=[
            pltpu.VMEM((KCH, EB), jnp.int32),
            pltpu.VMEM((EB, F16), jnp.float32),
            pltpu.VMEM_SHARED((NPAD, F16), jnp.float32),
            pltpu.SemaphoreType.DMA,
        ],
        compiler_params=pltpu.CompilerParams(use_tc_tiling_on_sc=False),
    )
    def deg_kernel(dst_hbm, ones_hbm, zeros_hbm, out_hbm, dstv, onesv, zsh,
                   ssem):
        c = lax.axis_index("c")
        s = lax.axis_index("s")
        w = _sc_worker_id()
        pltpu.sync_copy(ones_hbm, onesv)
        _sc_zero_table(zsh, zeros_hbm, s)
        plsc.subcore_barrier()

        base = w * RW

        def chunk(m, carry):
            r0 = base + m * KCH
            pltpu.sync_copy(dst_hbm.at[pl.ds(r0, KCH)], dstv)
            hs = [pltpu.async_copy(onesv, zsh.at[dstv.at[j]], ssem, add=True)
                  for j in range(KCH)]
            for h in hs:
                h.wait()
            return carry
        lax.fori_loop(0, NCHUNK, chunk, 0)

        plsc.subcore_barrier()
        _sc_copy_out(zsh, out_hbm, c, s)

    return deg_kernel


@functools.cache
def _make_prop_kernel(NPAD, EPR):
    RW = EPR // NW
    NCHUNK = RW // KCH
    mesh = plsc.VectorSubcoreMesh(core_axis_name="c", subcore_axis_name="s")

    @functools.partial(
        pl.kernel,
        out_type=jax.ShapeDtypeStruct((NC, NPAD, F16), jnp.float32),
        mesh=mesh,
        scratch_types=[
            pltpu.VMEM((KCH, EB), jnp.int32),
            pltpu.VMEM((KCH, EB), jnp.int32),
            pltpu.VMEM((NBUF, EB, F16), jnp.float32),
            pltpu.VMEM_SHARED((NPAD, F16), jnp.float32),
            pltpu.SemaphoreType.DMA,
            pltpu.SemaphoreType.DMA,
        ],
        compiler_params=pltpu.CompilerParams(use_tc_tiling_on_sc=False),
    )
    def prop_kernel(y_hbm, src_hbm, dst_hbm, zeros_hbm, out_hbm,
                    srcv, dstv, rows, zsh, gsem, ssem):
        c = lax.axis_index("c")
        s = lax.axis_index("s")
        w = _sc_worker_id()
        _sc_zero_table(zsh, zeros_hbm, s)
        plsc.subcore_barrier()

        base = w * RW

        def chunk(m, carry):
            r0 = base + m * KCH
            pltpu.sync_copy(src_hbm.at[pl.ds(r0, KCH)], srcv)
            pltpu.sync_copy(dst_hbm.at[pl.ds(r0, KCH)], dstv)
            gp = [None] * NBUF
            sp = [None] * NBUF
            for j in range(KCH):
                b = j % NBUF
                if sp[b] is not None:
                    sp[b].wait()
                    sp[b] = None
                gp[b] = pltpu.async_copy(
                    y_hbm.at[srcv.at[j]], rows.at[b], gsem)
                pb = (j - 1) % NBUF
                if j > 0:
                    gp[pb].wait()
                    gp[pb] = None
                    sp[pb] = pltpu.async_copy(
                        rows.at[pb], zsh.at[dstv.at[j - 1]], ssem, add=True)
            lb = (KCH - 1) % NBUF
            gp[lb].wait()
            sp[lb] = pltpu.async_copy(
                rows.at[lb], zsh.at[dstv.at[KCH - 1]], ssem, add=True)
            for b in range(NBUF):
                if sp[b] is not None:
                    sp[b].wait()
            return carry
        lax.fori_loop(0, NCHUNK, chunk, 0)

        plsc.subcore_barrier()
        _sc_copy_out(zsh, out_hbm, c, s)

    return prop_kernel


def _edge_pipeline(y_hbm, src_hbm, dst_hbm, zsh, srcv, dstv, rows,
                   gsem, ssem, base, NCHUNK):
    def chunk(m, carry):
        r0 = base + m * KCH
        pltpu.sync_copy(src_hbm.at[pl.ds(r0, KCH)], srcv)
        pltpu.sync_copy(dst_hbm.at[pl.ds(r0, KCH)], dstv)
        gp = [None] * NBUF
        sp = [None] * NBUF
        for j in range(KCH):
            b = j % NBUF
            if sp[b] is not None:
                sp[b].wait()
                sp[b] = None
            gp[b] = pltpu.async_copy(y_hbm.at[srcv.at[j]], rows.at[b], gsem)
            pb = (j - 1) % NBUF
            if j > 0:
                gp[pb].wait()
                gp[pb] = None
                sp[pb] = pltpu.async_copy(
                    rows.at[pb], zsh.at[dstv.at[j - 1]], ssem, add=True)
        lb = (KCH - 1) % NBUF
        gp[lb].wait()
        sp[lb] = pltpu.async_copy(
            rows.at[lb], zsh.at[dstv.at[KCH - 1]], ssem, add=True)
        for b in range(NBUF):
            if sp[b] is not None:
                sp[b].wait()
        return carry
    lax.fori_loop(0, NCHUNK, chunk, 0)


@functools.cache
def _make_dualprop_kernel(NPAD, EPR):
    RW = EPR // NS
    NCHUNK = RW // KCH
    mesh = plsc.VectorSubcoreMesh(core_axis_name="c", subcore_axis_name="s")

    @functools.partial(
        pl.kernel,
        out_type=[jax.ShapeDtypeStruct((NPAD, F16), jnp.float32),
                  jax.ShapeDtypeStruct((NPAD, F16), jnp.float32)],
        mesh=mesh,
        scratch_types=[
            pltpu.VMEM((KCH, EB), jnp.int32),
            pltpu.VMEM((KCH, EB), jnp.int32),
            pltpu.VMEM((NBUF, EB, F16), jnp.float32),
            pltpu.VMEM_SHARED((NPAD, F16), jnp.float32),
            pltpu.SemaphoreType.DMA,
            pltpu.SemaphoreType.DMA,
        ],
        compiler_params=pltpu.CompilerParams(use_tc_tiling_on_sc=False),
    )
    def dual_kernel(ya_hbm, yb_hbm, src_hbm, dst_hbm, zeros_hbm,
                    outa_hbm, outb_hbm, srcv, dstv, rows, zsh, gsem, ssem):
        c = lax.axis_index("c")
        s = lax.axis_index("s")
        _sc_zero_table(zsh, zeros_hbm, s)
        plsc.subcore_barrier()

        base = s * RW

        @pl.when(c == 0)
        def _():
            _edge_pipeline(ya_hbm, src_hbm, dst_hbm, zsh, srcv, dstv, rows,
                           gsem, ssem, base, NCHUNK)

        @pl.when(c == 1)
        def _():
            _edge_pipeline(yb_hbm, src_hbm, dst_hbm, zsh, srcv, dstv, rows,
                           gsem, ssem, base, NCHUNK)

        plsc.subcore_barrier()

        @pl.when(jnp.logical_and(s == 0, c == 0))
        def _():
            pltpu.sync_copy(zsh, outa_hbm)

        @pl.when(jnp.logical_and(s == 0, c == 1))
        def _():
            pltpu.sync_copy(zsh, outb_hbm)

    return dual_kernel


def _dinv_from_deg(degp):
    deg = degp[0, :, 0:1] + degp[1, :, 0:1] + 1.0
    return lax.rsqrt(deg)


def _prep_body(degp_ref, x_ref, ya_ref, yb_ref):
    dinv = _dinv_from_deg(degp_ref[...])
    y = x_ref[...] * dinv
    ya_ref[...] = y[:, :F16]
    yb_ref[...] = y[:, F16:]


def _stage_b_body(z1a_ref, z1b_ref, y1a_ref, y1b_ref, degp_ref,
                  w1_ref, b1_ref, w2_ref, *out_refs):
    dinv = _dinv_from_deg(degp_ref[...])
    u1a = dinv * (z1a_ref[...] + y1a_ref[...])
    u1b = dinv * (z1b_ref[...] + y1b_ref[...])
    u1 = jnp.concatenate([u1a, u1b], axis=1)
    h1 = jax.nn.relu(jnp.dot(u1, w1_ref[...],
                             preferred_element_type=jnp.float32) + b1_ref[...])
    g = jnp.dot(h1, w2_ref[...], preferred_element_type=jnp.float32)
    y2 = g * dinv
    for j, oref in enumerate(out_refs):
        oref[...] = y2[:, F16 * j:F16 * (j + 1)]


def _stage_c_body(*refs):
    (z0, z1, z2, z3, z4, y0, y1, y2, y3, y4, degp_ref,
     b2_ref, wih_ref, whh_ref, bg_ref, wfc_ref, bfc_ref, out_ref) = refs
    dinv = _dinv_from_deg(degp_ref[...])
    parts = []
    for zr, yr in ((z0, y0), (z1, y1), (z2, y2), (z3, y3), (z4, y4)):
        parts.append(dinv * (zr[...] + yr[...]))
    u2 = jnp.concatenate(parts, axis=1)
    gnn = jax.nn.relu(u2 + b2_ref[...])
    wih = wih_ref[...]
    whh = whh_ref[...]
    bg = bg_ref[...]
    n = gnn.shape[0]
    h = jnp.zeros((n, 16), jnp.float32)
    c = jnp.zeros((n, 16), jnp.float32)
    for t in range(10):
        xt = gnn[:, 8 * t:8 * t + 8]
        gates = (jnp.dot(xt, wih, preferred_element_type=jnp.float32)
                 + jnp.dot(h, whh, preferred_element_type=jnp.float32) + bg)
        i = jax.nn.sigmoid(gates[:, 0:16])
        f = jax.nn.sigmoid(gates[:, 16:32])
        g = jnp.tanh(gates[:, 32:48])
        o = jax.nn.sigmoid(gates[:, 48:64])
        c = f * c + i * g
        h = o * jnp.tanh(c)
    pred = jnp.sum(h * wfc_ref[...], axis=1, keepdims=True) + bfc_ref[...][:, 0:1]
    out_ref[...] = pred


def _node_spec(width):
    return pl.BlockSpec((BN, width), lambda i: (i, 0))


def _part_spec():
    return pl.BlockSpec((NC, BN, F16), lambda i: (0, i, 0))


def _full_spec(shape):
    return pl.BlockSpec(shape, lambda i: tuple(0 for _ in shape))


@functools.cache
def _make_prep_call(NPAD):
    return pl.pallas_call(
        _prep_body,
        grid=(NPAD // BN,),
        in_specs=[_part_spec(), _node_spec(32)],
        out_specs=[_node_spec(F16), _node_spec(F16)],
        out_shape=[jax.ShapeDtypeStruct((NPAD, F16), jnp.float32)] * 2,
    )


@functools.cache
def _make_stage_b_call(NPAD):
    return pl.pallas_call(
        _stage_b_body,
        grid=(NPAD // BN,),
        in_specs=[_node_spec(F16), _node_spec(F16),
                  _node_spec(F16), _node_spec(F16), _part_spec(),
                  _full_spec((32, 160)), _full_spec((1, 160)),
                  _full_spec((160, 80))],
        out_specs=[_node_spec(F16)] * 5,
        out_shape=[jax.ShapeDtypeStruct((NPAD, F16), jnp.float32)] * 5,
    )


@functools.cache
def _make_stage_c_call(NPAD):
    return pl.pallas_call(
        _stage_c_body,
        grid=(NPAD // BN,),
        in_specs=[_node_spec(F16)] * 5 + [_node_spec(F16)] * 5
        + [_part_spec(),
           _full_spec((1, 80)), _full_spec((8, 64)), _full_spec((16, 64)),
           _full_spec((1, 64)), _full_spec((1, 16)), _full_spec((1, 16))],
        out_specs=_node_spec(1),
        out_shape=jax.ShapeDtypeStruct((NPAD, 1), jnp.float32),
    )


def kernel(x_seq, edge_index, W1, b1, W2, b2, W_ih, W_hh, b_ih, b_hh,
           W_fc, b_fc):
    T, N, FIN = x_seq.shape
    E = edge_index.shape[1]
    NPAD = _cdiv(N + 1, NS * 128) * NS * 128
    EPG = EB * KCH * NW
    EPR = _cdiv(E, EPG) * KCH * NW
    EP = EPR * EB

    src = jnp.concatenate(
        [edge_index[0], jnp.zeros((EP - E,), jnp.int32)]).reshape(EPR, EB)
    dst = jnp.concatenate(
        [edge_index[1], jnp.full((EP - E,), N, jnp.int32)]).reshape(EPR, EB)
    ones_t = jnp.ones((EB, F16), jnp.float32)
    zeros_t = jnp.zeros((NPAD, F16), jnp.float32)

    degp = _make_deg_kernel(NPAD, EPR)(dst, ones_t, zeros_t)

    x2 = jnp.transpose(x_seq, (1, 0, 2)).reshape(N, T * FIN)
    x2p = jnp.pad(x2, ((0, NPAD - N), (0, 32 - T * FIN)))

    y1a, y1b = _make_prep_call(NPAD)(degp, x2p)

    propk = _make_prop_kernel(NPAD, EPR)
    dualk = _make_dualprop_kernel(NPAD, EPR)

    def chain(z_prev):
        return zeros_t + z_prev.reshape(-1)[0] * 0.0

    z1a, z1b = dualk(y1a, y1b, src, dst, chain(degp))

    W1big = jax.scipy.linalg.block_diag(*([W1] * T))
    W1big = jnp.pad(W1big, ((0, 32 - T * FIN), (0, 0)))
    b1big = jnp.tile(b1, T)[None, :]
    W2big = jax.scipy.linalg.block_diag(*([W2] * T))

    y2s = _make_stage_b_call(NPAD)(z1a, z1b, y1a, y1b, degp,
                                   W1big, b1big, W2big)

    z2_0, z2_1 = dualk(y2s[0], y2s[1], src, dst, chain(z1a))
    z2_2, z2_3 = dualk(y2s[2], y2s[3], src, dst, chain(z2_0))
    z2_4p = propk(y2s[4], src, dst, chain(z2_2))
    z2_4 = z2_4p[0] + z2_4p[1]
    z2s = [z2_0, z2_1, z2_2, z2_3, z2_4]

    b2big = jnp.tile(b2, T)[None, :]
    bg = (b_ih + b_hh)[None, :]
    wfc_row = jnp.broadcast_to(W_fc.reshape(1, 16), (1, 16))
    bfc = jnp.broadcast_to(b_fc.reshape(1, 1), (1, 16))

    pred_full = _make_stage_c_call(NPAD)(
        *z2s, *y2s, degp, b2big, W_ih.T, W_hh.T, bg, wfc_row, bfc)
    return pred_full[:N]
